# Initial kernel scaffold; baseline (speedup 1.0000x reference)
import jax
import jax.numpy as jnp
from jax.experimental import pallas as pl


def kernel(logits, sem_gt, is_train, previous_features, previous_count, var):
    raise NotImplementedError("write your pallas kernel here")


# speedup vs baseline: 1.7391x; 1.7391x over previous
import functools

import jax
import jax.numpy as jnp
from jax.experimental import pallas as pl
from jax.experimental.pallas import tpu as pltpu

_NC = 19
_SMOOTH = 0.01
_DELTA = 0.1


def _tile_body(x_ref, g_ref, a_ref, s_ref, sums_ref, cnts_ref):
    @pl.when(pl.program_id(0) == 0)
    def _init():
        sums_ref[...] = jnp.zeros_like(sums_ref)
        cnts_ref[...] = jnp.zeros_like(cnts_ref)

    g = g_ref[0]
    for l in range(1, _NC):
        mask = (g == l).astype(jnp.float32)
        acc = None
        for c in range(_NC):
            t = jnp.abs(x_ref[0, c] - a_ref[l, c]) * s_ref[l, c] - _DELTA
            t = jnp.maximum(t, 0.0)
            acc = t if acc is None else acc + t
        masked = acc * mask
        sums_ref[l, :] += jnp.sum(masked, axis=0)
        cnts_ref[l, :] += jnp.sum(mask, axis=0)


@functools.partial(jax.jit, static_argnames=("bh",))
def _owloss_tc(logits, sem_gt, a_tab, s_tab, bh=16):
    B, C, H, W = logits.shape
    nh = H // bh
    grid = (B * nh,)
    out = pl.pallas_call(
        _tile_body,
        grid=grid,
        in_specs=[
            pl.BlockSpec((1, C, bh, W), lambda i: (i // nh, 0, i % nh, 0)),
            pl.BlockSpec((1, bh, W), lambda i: (i // nh, i % nh, 0)),
            pl.BlockSpec(memory_space=pltpu.SMEM),
            pl.BlockSpec(memory_space=pltpu.SMEM),
        ],
        out_specs=[
            pl.BlockSpec((24, W), lambda i: (0, 0)),
            pl.BlockSpec((24, W), lambda i: (0, 0)),
        ],
        out_shape=[
            jax.ShapeDtypeStruct((24, W), jnp.float32),
            jax.ShapeDtypeStruct((24, W), jnp.float32),
        ],
    )(logits, sem_gt, a_tab, s_tab)
    return out


def kernel(logits, sem_gt, is_train, previous_features, previous_count, var):
    pos = var > 0
    absv = jnp.abs(var)
    nzmin = jnp.min(jnp.where(pos, absv, jnp.inf), axis=1, keepdims=True)
    variance = jnp.where(pos, nzmin, var)
    norm_var = variance / nzmin
    s_tab = 1.0 / (norm_var + _SMOOTH)
    a_tab = previous_features

    sums2d, cnts2d = _owloss_tc(logits, sem_gt.astype(jnp.int32), a_tab, s_tab)
    sums = jnp.sum(sums2d[: _NC], axis=1)
    cnts = jnp.sum(cnts2d[: _NC], axis=1)

    lbl = jnp.arange(_NC)
    denom = jnp.maximum(cnts * _NC, 1.0)
    mean_val = sums / denom
    cond = (lbl >= 1) & (cnts > 0) & (previous_count > 0) & (jnp.sum(var, axis=1) != 0)
    acc = jnp.sum(jnp.where(cond, mean_val, 0.0))
    return jnp.clip(acc, 0.0, 20.0)


# speedup vs baseline: 3.1455x; 1.8086x over previous
import functools

import jax
import jax.numpy as jnp
from jax import lax
from jax.experimental import pallas as pl
from jax.experimental.pallas import tpu as pltpu
from jax.experimental.pallas import tpu_sc as plsc

_NC = 19
_SMOOTH = 0.01
_DELTA = 0.1

_L = 16
_T = 2048
_BINS = _NC * _L


def _sc_body(nw, px_per_w, ppi, x_hbm, lab_hbm, a_hbm, s_hbm, parts_hbm,
             a_v, s_v, lab_v, x_v, acc_s, acc_c, sem_x0, sem_x1, sem_l0,
             sem_l1):
    ncores = 2
    wid = lax.axis_index("s") * ncores + lax.axis_index("c")
    g_base = wid * px_per_w
    b = g_base // ppi
    p_base = g_base % ppi

    pltpu.sync_copy(a_hbm, a_v)
    pltpu.sync_copy(s_hbm, s_v)

    zero16 = jnp.zeros((_L,), jnp.float32)
    for i in range(_NC):
        acc_s[pl.ds(i * _L, _L)] = zero16
        acc_c[pl.ds(i * _L, _L)] = zero16

    n_tiles = px_per_w // _T
    sems_x = (sem_x0, sem_x1)
    sems_l = (sem_l0, sem_l1)

    def tile_copies(t, buf):
        off = p_base + t * _T
        cx = pltpu.make_async_copy(
            x_hbm.at[b, :, pl.ds(off, _T)], x_v.at[buf], sems_x[buf])
        cl = pltpu.make_async_copy(
            lab_hbm.at[pl.ds(g_base + t * _T, _T)], lab_v.at[buf],
            sems_l[buf])
        return cx, cl

    def start_tile(t, buf):
        for c in tile_copies(t, buf):
            c.start()

    def wait_tile(t, buf):
        for c in tile_copies(t, buf):
            c.wait()

    start_tile(0, 0)

    iota16 = lax.iota(jnp.int32, _L)
    ones16 = jnp.ones((_L,), jnp.float32)

    for t in range(n_tiles):
        buf = t % 2
        if t + 1 < n_tiles:
            start_tile(t + 1, (t + 1) % 2)
        wait_tile(t, buf)

        def vec_body(v, carry, buf=buf):
            base = v * _L
            lab16 = lab_v[buf, pl.ds(base, _L)]
            idx = lab16
            y = zero16
            for c in range(_NC):
                x = x_v[buf, c, pl.ds(base, _L)]
                a = plsc.load_gather(a_v, [idx])
                s = plsc.load_gather(s_v, [idx])
                y = y + jnp.maximum(jnp.abs(x - a) * s - _DELTA, 0.0)
                if c + 1 < _NC:
                    idx = idx + _NC
            sidx = lab16 * _L + iota16
            plsc.addupdate_scatter(acc_s, [sidx], y)
            plsc.addupdate_scatter(acc_c, [sidx], ones16)
            return carry

        lax.fori_loop(0, _T // _L, vec_body, 0)

    pltpu.sync_copy(acc_s, parts_hbm.at[wid, 0])
    pltpu.sync_copy(acc_c, parts_hbm.at[wid, 1])


@jax.jit
def _owloss_sc(x3, lab, a_tab, s_tab):
    mesh = plsc.VectorSubcoreMesh(core_axis_name="c", subcore_axis_name="s")
    nw = mesh.num_cores * mesh.num_subcores
    npix = lab.shape[0]
    px_per_w = npix // nw
    ppi = x3.shape[2]
    body = functools.partial(_sc_body, nw, px_per_w, ppi)
    parts = pl.kernel(
        body,
        out_type=jax.ShapeDtypeStruct((nw, 2, _BINS), jnp.float32),
        mesh=mesh,
        compiler_params=pltpu.CompilerParams(
            needs_layout_passes=False, use_tc_tiling_on_sc=False),
        scratch_types=[
            pltpu.VMEM((a_tab.shape[0],), jnp.float32),
            pltpu.VMEM((s_tab.shape[0],), jnp.float32),
            pltpu.VMEM((2, _T), jnp.int32),
            pltpu.VMEM((2, _NC, _T), jnp.float32),
            pltpu.VMEM((_BINS,), jnp.float32),
            pltpu.VMEM((_BINS,), jnp.float32),
            pltpu.SemaphoreType.DMA,
            pltpu.SemaphoreType.DMA,
            pltpu.SemaphoreType.DMA,
            pltpu.SemaphoreType.DMA,
        ],
    )(x3, lab, a_tab, s_tab)
    return parts


def kernel(logits, sem_gt, is_train, previous_features, previous_count, var):
    B, C, H, W = logits.shape
    pos = var > 0
    absv = jnp.abs(var)
    nzmin = jnp.min(jnp.where(pos, absv, jnp.inf), axis=1, keepdims=True)
    variance = jnp.where(pos, nzmin, var)
    norm_var = variance / nzmin
    s_full = 1.0 / (norm_var + _SMOOTH)

    pad = (-(_NC * _NC)) % 8
    a_tab = jnp.pad(previous_features.T.reshape(-1), (0, pad))
    s_tab = jnp.pad(s_full.T.reshape(-1), (0, pad))

    x3 = logits.reshape(B, C, H * W)
    lab = sem_gt.astype(jnp.int32).reshape(-1)

    parts = _owloss_sc(x3, lab, a_tab, s_tab)
    sums = parts[:, 0, :].reshape(-1, _NC, _L).sum(axis=(0, 2))
    cnts = parts[:, 1, :].reshape(-1, _NC, _L).sum(axis=(0, 2))

    lbl = jnp.arange(_NC)
    denom = jnp.maximum(cnts * _NC, 1.0)
    mean_val = sums / denom
    cond = (lbl >= 1) & (cnts > 0) & (previous_count > 0) & (jnp.sum(var, axis=1) != 0)
    acc = jnp.sum(jnp.where(cond, mean_val, 0.0))
    return jnp.clip(acc, 0.0, 20.0)
